# baseline (speedup 1.0000x reference)
import functools

import jax
import jax.numpy as jnp
from jax import lax
from jax.experimental import pallas as pl
from jax.experimental.pallas import tpu as pltpu
from jax.experimental.pallas import tpu_sc as plsc

_NC = 2
_NS = 16
_NW = _NC * _NS
_L = 16
_IDXW = 128
_WC_POS = 16
_WC_NEG = 8


@functools.lru_cache(maxsize=None)
def _build(V, D, B, P, N):
  wpw = B // _NW
  cr_pos = _WC_POS * P
  cr_neg = _WC_NEG * N
  cr_max = max(cr_pos, cr_neg)
  idx_max = wpw * max(P, N)

  mesh = plsc.VectorSubcoreMesh(
      core_axis_name="c", subcore_axis_name="s",
      num_cores=_NC, num_subcores=_NS)

  @functools.partial(
      pl.kernel,
      out_type=(jax.ShapeDtypeStruct((B * P,), jnp.float32),
                jax.ShapeDtypeStruct((B * N,), jnp.float32)),
      mesh=mesh,
      compiler_params=pltpu.CompilerParams(use_tc_tiling_on_sc=False,
                                           needs_layout_passes=False),
      scratch_types=[
          pltpu.VMEM((wpw,), jnp.int32),
          pltpu.VMEM((wpw, D), jnp.bfloat16),
          pltpu.VMEM((idx_max,), jnp.int32),
          pltpu.VMEM((cr_max, D), jnp.bfloat16),
          pltpu.VMEM((cr_max, D), jnp.bfloat16),
          pltpu.VMEM((cr_max,), jnp.float32),
          pltpu.VMEM((cr_max,), jnp.float32),
          pltpu.VMEM((cr_max * 17,), jnp.float32),
          pltpu.SemaphoreType.DMA,
          pltpu.SemaphoreType.DMA,
          pltpu.SemaphoreType.DMA,
          pltpu.SemaphoreType.DMA,
      ],
  )
  def run(wt_hbm, ct_hbm, words_hbm, pos_hbm, neg_hbm,
          out_p_hbm, out_n_hbm,
          widx_v, wrows_v, cidx_v, crows0, crows1, out0, out1,
          psum_v, semg0, semg1, semo0, semo1):
    wid = lax.axis_index("s") * _NC + lax.axis_index("c")
    lane17 = lax.iota(jnp.int32, _L) * 17

    def dec(x32):
      u = plsc.bitcast(x32, jnp.uint32)
      lo = plsc.bitcast(u << jnp.uint32(16), jnp.float32)
      hi = plsc.bitcast(u & jnp.uint32(0xFFFF0000), jnp.float32)
      return lo, hi

    pltpu.sync_copy(words_hbm.at[pl.ds(wid * wpw, wpw)], widx_v)
    wcps = [
        pltpu.async_copy(wt_hbm.at[widx_v.at[pl.ds(j * _IDXW, _IDXW)]],
                         wrows_v.at[pl.ds(j * _IDXW, _IDXW)], semg0)
        for j in range(wpw // _IDXW)
    ]
    for cp in wcps:
      cp.wait()

    def do_pass(ctx_idx_hbm, out_hbm, wc, reps):
      cr = wc * reps
      n_chunks = wpw // wc
      base = wid * wpw * reps
      crows = (crows0, crows1)
      outs = (out0, out1)
      semg = (semg0, semg1)
      semo = (semo0, semo1)
      n_full = cr // _IDXW
      rem = cr - n_full * _IDXW
      dmas = [(j * _IDXW, _IDXW) for j in range(n_full)]
      if rem:
        dmas.append((n_full * _IDXW, rem))

      pltpu.sync_copy(ctx_idx_hbm.at[pl.ds(base, wpw * reps)],
                      cidx_v.at[pl.ds(0, wpw * reps)])

      def fire(cc, b):
        for (o, n) in dmas:
          pltpu.async_copy(
              ct_hbm.at[cidx_v.at[pl.ds(cc * cr + o, n)]],
              crows[b].at[pl.ds(o, n)], semg[b])

      def drain(b):
        for (o, n) in dmas:
          pltpu.make_async_copy(
              ct_hbm.at[cidx_v.at[pl.ds(o, n)]],
              crows[b].at[pl.ds(o, n)], semg[b]).wait()

      def compute(c, b):
        cstart = c * wc

        def word_body(iw, _):
          wr = []
          for k in range(D // (2 * _L)):
            wr += list(dec(wrows_v[cstart + iw, pl.ds(k * 2 * _L, 2 * _L)]))
          for j in range(reps):
            r = iw * reps + j
            ts = []
            for k in range(D // (2 * _L)):
              lo, hi = dec(crows[b][r, pl.ds(k * 2 * _L, 2 * _L)])
              ts += [wr[2 * k] * lo, wr[2 * k + 1] * hi]
            while len(ts) > 1:
              ts = [ts[i] + ts[i + 1] for i in range(0, len(ts) - 1, 2)] + (
                  [ts[-1]] if len(ts) % 2 else [])
            psum_v[pl.ds(r * 17, _L)] = ts[0]
          return 0

        lax.fori_loop(0, wc, word_body, 0)

        def red_body(g, _):
          va = lane17 + g * (17 * _L)
          cols = [plsc.load_gather(psum_v, [va + d]) for d in range(_L)]
          while len(cols) > 1:
            cols = [cols[i] + cols[i + 1] for i in range(0, len(cols), 2)]
          outs[b][pl.ds(g * _L, _L)] = 1.0 / (1.0 + jnp.exp(-cols[0]))
          return 0

        lax.fori_loop(0, cr // _L, red_body, 0)

      def out_wait(b):
        pltpu.make_async_copy(outs[b].at[pl.ds(0, cr)],
                              out_hbm.at[pl.ds(base, cr)], semo[b]).wait()

      fire(0, 0)
      fire(1, 1)
      pltpu.async_copy(out_hbm.at[pl.ds(base, cr)], outs[0].at[pl.ds(0, cr)],
                       semo[0])
      pltpu.async_copy(out_hbm.at[pl.ds(base, cr)], outs[1].at[pl.ds(0, cr)],
                       semo[1])

      @pl.loop(0, n_chunks, step=2)
      def _(c):
        for b in range(2):
          drain(b)
          out_wait(b)
          compute(c + b, b)
          pltpu.async_copy(outs[b].at[pl.ds(0, cr)],
                           out_hbm.at[pl.ds(base + (c + b) * cr, cr)],
                           semo[b])
          fire(jnp.minimum(c + 2 + b, n_chunks - 1), b)

      drain(0)
      drain(1)
      out_wait(0)
      out_wait(1)

    do_pass(pos_hbm, out_p_hbm, _WC_POS, P)
    do_pass(neg_hbm, out_n_hbm, _WC_NEG, N)

  return run


def kernel(word_table, ctx_table, words, positive_contexts, negative_contexts):
  V, D = word_table.shape
  B = words.shape[0]
  P = positive_contexts.shape[1]
  N = negative_contexts.shape[1]
  run = _build(V, D, B, P, N)
  return run(word_table.astype(jnp.bfloat16), ctx_table.astype(jnp.bfloat16),
             words.astype(jnp.int32),
             positive_contexts.astype(jnp.int32).reshape(B * P),
             negative_contexts.astype(jnp.int32).reshape(B * N))


# speedup vs baseline: 1.2184x; 1.2184x over previous
import functools

import jax
import jax.numpy as jnp
from jax import lax
from jax.experimental import pallas as pl
from jax.experimental.pallas import tpu as pltpu
from jax.experimental.pallas import tpu_sc as plsc

_NC = 2
_NS = 16
_NW = _NC * _NS
_L = 16
_IDXW = 128
_WC_POS = 16
_WC_NEG = 8

_PARAMS = pltpu.CompilerParams(use_tc_tiling_on_sc=False,
                               needs_layout_passes=False)


@functools.lru_cache(maxsize=None)
def _build_word_gather(V, D, B):
  wpw = B // _NW
  mesh = plsc.VectorSubcoreMesh(
      core_axis_name="c", subcore_axis_name="s",
      num_cores=_NC, num_subcores=_NS)

  @functools.partial(
      pl.kernel,
      out_type=jax.ShapeDtypeStruct((B, D), jnp.float32),
      mesh=mesh,
      compiler_params=_PARAMS,
      scratch_types=[
          pltpu.VMEM((wpw,), jnp.int32),
          pltpu.VMEM((wpw, D), jnp.float32),
          pltpu.SemaphoreType.DMA,
      ],
  )
  def run(wt_hbm, words_hbm, wout_hbm, widx_v, wbuf_v, sem):
    wid = lax.axis_index("s") * _NC + lax.axis_index("c")
    pltpu.sync_copy(words_hbm.at[pl.ds(wid * wpw, wpw)], widx_v)
    cps = [
        pltpu.async_copy(wt_hbm.at[widx_v.at[pl.ds(j * _IDXW, _IDXW)]],
                         wbuf_v.at[pl.ds(j * _IDXW, _IDXW)], sem)
        for j in range(wpw // _IDXW)
    ]
    for cp in cps:
      cp.wait()
    pltpu.sync_copy(wbuf_v, wout_hbm.at[pl.ds(wid * wpw, wpw)])

  return run


@functools.lru_cache(maxsize=None)
def _build_main(V, D, B, P, N):
  wpw = B // _NW
  cr_pos = _WC_POS * P
  cr_neg = _WC_NEG * N
  cr_max = max(cr_pos, cr_neg)
  idx_max = wpw * max(P, N)

  mesh = plsc.VectorSubcoreMesh(
      core_axis_name="c", subcore_axis_name="s",
      num_cores=_NC, num_subcores=_NS)

  @functools.partial(
      pl.kernel,
      out_type=(jax.ShapeDtypeStruct((B * P,), jnp.float32),
                jax.ShapeDtypeStruct((B * N,), jnp.float32)),
      mesh=mesh,
      compiler_params=_PARAMS,
      scratch_types=[
          pltpu.VMEM((wpw, D), jnp.float32),
          pltpu.VMEM((idx_max,), jnp.int32),
          pltpu.VMEM((cr_max, D), jnp.float32),
          pltpu.VMEM((cr_max, D), jnp.float32),
          pltpu.VMEM((cr_max,), jnp.float32),
          pltpu.VMEM((cr_max,), jnp.float32),
          pltpu.VMEM((cr_max * 17,), jnp.float32),
          pltpu.SemaphoreType.DMA,
          pltpu.SemaphoreType.DMA,
          pltpu.SemaphoreType.DMA,
          pltpu.SemaphoreType.DMA,
      ],
  )
  def run(wrows_hbm, ct_hbm, pos_hbm, neg_hbm,
          out_p_hbm, out_n_hbm,
          wrows_v, cidx_v, crows0, crows1, out0, out1,
          psum_v, semg0, semg1, semo0, semo1):
    wid = lax.axis_index("s") * _NC + lax.axis_index("c")
    lane17 = lax.iota(jnp.int32, _L) * 17

    pltpu.sync_copy(wrows_hbm.at[pl.ds(wid * wpw, wpw)], wrows_v)

    def do_pass(ctx_idx_hbm, out_hbm, wc, reps):
      cr = wc * reps
      n_chunks = wpw // wc
      base = wid * wpw * reps
      crows = (crows0, crows1)
      outs = (out0, out1)
      semg = (semg0, semg1)
      semo = (semo0, semo1)
      n_full = cr // _IDXW
      rem = cr - n_full * _IDXW
      dmas = [(j * _IDXW, _IDXW) for j in range(n_full)]
      if rem:
        dmas.append((n_full * _IDXW, rem))

      pltpu.sync_copy(ctx_idx_hbm.at[pl.ds(base, wpw * reps)],
                      cidx_v.at[pl.ds(0, wpw * reps)])

      def fire(cc, b):
        for (o, n) in dmas:
          pltpu.async_copy(
              ct_hbm.at[cidx_v.at[pl.ds(cc * cr + o, n)]],
              crows[b].at[pl.ds(o, n)], semg[b])

      def drain(b):
        for (o, n) in dmas:
          pltpu.make_async_copy(
              ct_hbm.at[cidx_v.at[pl.ds(o, n)]],
              crows[b].at[pl.ds(o, n)], semg[b]).wait()

      def compute(c, b):
        cstart = c * wc

        def word_body(iw, _):
          wr = [wrows_v[cstart + iw, pl.ds(k * _L, _L)]
                for k in range(D // _L)]
          for j in range(reps):
            r = iw * reps + j
            ts = [wr[k] * crows[b][r, pl.ds(k * _L, _L)]
                  for k in range(D // _L)]
            while len(ts) > 1:
              ts = [ts[i] + ts[i + 1] for i in range(0, len(ts) - 1, 2)] + (
                  [ts[-1]] if len(ts) % 2 else [])
            psum_v[pl.ds(r * 17, _L)] = ts[0]
          return 0

        lax.fori_loop(0, wc, word_body, 0)

        def red_body(g, _):
          va = lane17 + g * (17 * _L)
          cols = [plsc.load_gather(psum_v, [va + d]) for d in range(_L)]
          while len(cols) > 1:
            cols = [cols[i] + cols[i + 1] for i in range(0, len(cols), 2)]
          outs[b][pl.ds(g * _L, _L)] = 1.0 / (1.0 + jnp.exp(-cols[0]))
          return 0

        lax.fori_loop(0, cr // _L, red_body, 0)

      def out_wait(b):
        pltpu.make_async_copy(outs[b].at[pl.ds(0, cr)],
                              out_hbm.at[pl.ds(base, cr)], semo[b]).wait()

      fire(0, 0)
      fire(1, 1)
      pltpu.async_copy(out_hbm.at[pl.ds(base, cr)], outs[0].at[pl.ds(0, cr)],
                       semo[0])
      pltpu.async_copy(out_hbm.at[pl.ds(base, cr)], outs[1].at[pl.ds(0, cr)],
                       semo[1])

      @pl.loop(0, n_chunks, step=2)
      def _(c):
        for b in range(2):
          drain(b)
          out_wait(b)
          compute(c + b, b)
          pltpu.async_copy(outs[b].at[pl.ds(0, cr)],
                           out_hbm.at[pl.ds(base + (c + b) * cr, cr)],
                           semo[b])
          fire(jnp.minimum(c + 2 + b, n_chunks - 1), b)

      drain(0)
      drain(1)
      out_wait(0)
      out_wait(1)

    do_pass(pos_hbm, out_p_hbm, _WC_POS, P)
    do_pass(neg_hbm, out_n_hbm, _WC_NEG, N)

  return run


def kernel(word_table, ctx_table, words, positive_contexts, negative_contexts):
  V, D = word_table.shape
  B = words.shape[0]
  P = positive_contexts.shape[1]
  N = negative_contexts.shape[1]
  wrows = _build_word_gather(V, D, B)(word_table, words.astype(jnp.int32))
  return _build_main(V, D, B, P, N)(
      wrows, ctx_table,
      positive_contexts.astype(jnp.int32).reshape(B * P),
      negative_contexts.astype(jnp.int32).reshape(B * N))


# speedup vs baseline: 1.2392x; 1.0171x over previous
import functools

import jax
import jax.numpy as jnp
from jax import lax
from jax.experimental import pallas as pl
from jax.experimental.pallas import tpu as pltpu
from jax.experimental.pallas import tpu_sc as plsc

_NC = 2
_NS = 16
_NW = _NC * _NS
_L = 16
_IDXW = 128
_WC_POS = 16
_WC_NEG = 8


@functools.lru_cache(maxsize=None)
def _build(V, D, B, P, N):
  wpw = B // _NW
  cr_pos = _WC_POS * P
  cr_neg = _WC_NEG * N
  cr_max = max(cr_pos, cr_neg)
  idx_max = wpw * max(P, N)

  mesh = plsc.VectorSubcoreMesh(
      core_axis_name="c", subcore_axis_name="s",
      num_cores=_NC, num_subcores=_NS)

  @functools.partial(
      pl.kernel,
      out_type=(jax.ShapeDtypeStruct((B * P,), jnp.float32),
                jax.ShapeDtypeStruct((B * N,), jnp.float32)),
      mesh=mesh,
      compiler_params=pltpu.CompilerParams(use_tc_tiling_on_sc=False,
                                           needs_layout_passes=False,
                                           skip_device_barrier=True),
      scratch_types=[
          pltpu.VMEM((wpw,), jnp.int32),
          pltpu.VMEM((wpw, D), jnp.float32),
          pltpu.VMEM((idx_max,), jnp.int32),
          pltpu.VMEM((cr_max, D), jnp.float32),
          pltpu.VMEM((cr_max, D), jnp.float32),
          pltpu.VMEM((cr_max,), jnp.float32),
          pltpu.VMEM((cr_max,), jnp.float32),
          pltpu.VMEM((cr_max * 17,), jnp.float32),
          pltpu.SemaphoreType.DMA,
          pltpu.SemaphoreType.DMA,
          pltpu.SemaphoreType.DMA,
          pltpu.SemaphoreType.DMA,
      ],
  )
  def run(wt_hbm, ct_hbm, words_hbm, pos_hbm, neg_hbm,
          out_p_hbm, out_n_hbm,
          widx_v, wrows_v, cidx_v, crows0, crows1, out0, out1,
          psum_v, semg0, semg1, semo0, semo1):
    wid = lax.axis_index("s") * _NC + lax.axis_index("c")
    lane17 = lax.iota(jnp.int32, _L) * 17

    pltpu.sync_copy(words_hbm.at[pl.ds(wid * wpw, wpw)], widx_v)
    wcps = [
        pltpu.async_copy(wt_hbm.at[widx_v.at[pl.ds(j * _IDXW, _IDXW)]],
                         wrows_v.at[pl.ds(j * _IDXW, _IDXW)], semg0)
        for j in range(wpw // _IDXW)
    ]
    for cp in wcps:
      cp.wait()

    def do_pass(ctx_idx_hbm, out_hbm, wc, reps):
      cr = wc * reps
      n_chunks = wpw // wc
      base = wid * wpw * reps
      crows = (crows0, crows1)
      outs = (out0, out1)
      semg = (semg0, semg1)
      semo = (semo0, semo1)
      n_full = cr // _IDXW
      rem = cr - n_full * _IDXW
      dmas = [(j * _IDXW, _IDXW) for j in range(n_full)]
      if rem:
        dmas.append((n_full * _IDXW, rem))

      pltpu.sync_copy(ctx_idx_hbm.at[pl.ds(base, wpw * reps)],
                      cidx_v.at[pl.ds(0, wpw * reps)])

      def fire(cc, b):
        for (o, n) in dmas:
          pltpu.async_copy(
              ct_hbm.at[cidx_v.at[pl.ds(cc * cr + o, n)]],
              crows[b].at[pl.ds(o, n)], semg[b])

      def drain(b):
        for (o, n) in dmas:
          pltpu.make_async_copy(
              ct_hbm.at[cidx_v.at[pl.ds(o, n)]],
              crows[b].at[pl.ds(o, n)], semg[b]).wait()

      def compute(c, b):
        cstart = c * wc

        def word_body(iw, _):
          wr = [wrows_v[cstart + iw, pl.ds(k * _L, _L)]
                for k in range(D // _L)]
          for j in range(reps):
            r = iw * reps + j
            ts = [wr[k] * crows[b][r, pl.ds(k * _L, _L)]
                  for k in range(D // _L)]
            while len(ts) > 1:
              ts = [ts[i] + ts[i + 1] for i in range(0, len(ts) - 1, 2)] + (
                  [ts[-1]] if len(ts) % 2 else [])
            psum_v[pl.ds(r * 17, _L)] = ts[0]
          return 0

        lax.fori_loop(0, wc, word_body, 0)

        def red_body(g, _):
          va = lane17 + g * (17 * _L)
          cols = [plsc.load_gather(psum_v, [va + d]) for d in range(_L)]
          while len(cols) > 1:
            cols = [cols[i] + cols[i + 1] for i in range(0, len(cols), 2)]
          outs[b][pl.ds(g * _L, _L)] = 1.0 / (1.0 + jnp.exp(-cols[0]))
          return 0

        lax.fori_loop(0, cr // _L, red_body, 0)

      def out_wait(b):
        pltpu.make_async_copy(outs[b].at[pl.ds(0, cr)],
                              out_hbm.at[pl.ds(base, cr)], semo[b]).wait()

      fire(0, 0)
      fire(1, 1)
      pltpu.async_copy(out_hbm.at[pl.ds(base, cr)], outs[0].at[pl.ds(0, cr)],
                       semo[0])
      pltpu.async_copy(out_hbm.at[pl.ds(base, cr)], outs[1].at[pl.ds(0, cr)],
                       semo[1])

      @pl.loop(0, n_chunks, step=2)
      def _(c):
        for b in range(2):
          drain(b)
          out_wait(b)
          compute(c + b, b)
          pltpu.async_copy(outs[b].at[pl.ds(0, cr)],
                           out_hbm.at[pl.ds(base + (c + b) * cr, cr)],
                           semo[b])
          fire(jnp.minimum(c + 2 + b, n_chunks - 1), b)

      drain(0)
      drain(1)
      out_wait(0)
      out_wait(1)

    do_pass(pos_hbm, out_p_hbm, _WC_POS, P)
    do_pass(neg_hbm, out_n_hbm, _WC_NEG, N)

  return run


def kernel(word_table, ctx_table, words, positive_contexts, negative_contexts):
  V, D = word_table.shape
  B = words.shape[0]
  P = positive_contexts.shape[1]
  N = negative_contexts.shape[1]
  run = _build(V, D, B, P, N)
  return run(word_table, ctx_table,
             words.astype(jnp.int32),
             positive_contexts.astype(jnp.int32).reshape(B * P),
             negative_contexts.astype(jnp.int32).reshape(B * N))
